# Initial kernel scaffold; baseline (speedup 1.0000x reference)
import jax
import jax.numpy as jnp
from jax.experimental import pallas as pl


def kernel(x, edge_index, batch, W1, b1, W2, b2, pool_w):
    raise NotImplementedError("write your pallas kernel here")


# speedup vs baseline: 15.6987x; 15.6987x over previous
import functools
import jax
import jax.numpy as jnp
from jax import lax
from jax.experimental import pallas as pl
from jax.experimental.pallas import tpu as pltpu
from jax.experimental.pallas import tpu_sc as plsc

N = 10000
E = 320000
F_IN = 128
H = 128
G = 64
RATIO = 0.5

NP = 10240
NC = 2
NS = 16
EPS = E // NC
EPW = E // (NC * NS)

f32 = jnp.float32
i32 = jnp.int32


def _fill_f32(ref, n, val):
    def body(i, _):
        ref[pl.ds(i * 16, 16)] = jnp.full((16,), val, f32)
        return 0
    lax.fori_loop(0, n // 16, body, 0)


_DEG_CH = 2000


def _deg_body(dst_hbm, out_hbm, deg_sh, dstv, ones, zbuf, sem):
    c = lax.axis_index("c")
    s = lax.axis_index("s")
    rows = NP // NS
    _fill_f32(zbuf, rows, 0.0)
    pltpu.sync_copy(zbuf, deg_sh.at[pl.ds(s * rows, rows)])
    _fill_f32(ones, _DEG_CH, 1.0)
    plsc.subcore_barrier()
    base = (c * NS + s) * EPW

    def chunk(j, _):
        pltpu.sync_copy(dst_hbm.at[pl.ds(base + j * _DEG_CH, _DEG_CH)], dstv)
        pltpu.sync_copy(ones, deg_sh.at[dstv], add=True)
        return 0
    lax.fori_loop(0, EPW // _DEG_CH, chunk, 0)
    plsc.subcore_barrier()
    pltpu.sync_copy(deg_sh.at[pl.ds(s * rows, rows)],
                    out_hbm.at[c, pl.ds(s * rows, rows)])


@functools.lru_cache(maxsize=1)
def _deg_call():
    return pl.kernel(
    _deg_body,
    out_type=jax.ShapeDtypeStruct((NC, NP), f32),
    mesh=plsc.VectorSubcoreMesh(core_axis_name="c", subcore_axis_name="s",
                                num_cores=NC, num_subcores=NS),
    scratch_types=[
        pltpu.VMEM_SHARED((NP,), f32),
        pltpu.VMEM((_DEG_CH,), i32),
        pltpu.VMEM((_DEG_CH,), f32),
        pltpu.VMEM((NP // NS,), f32),
        pltpu.SemaphoreType.DMA,
    ],
)


_C_CH = 2000


def _cmat_body(src_hbm, dst_hbm, x_hbm, dinv_hbm, out_hbm,
               c_sh, srcv, dstv, vv, fv, zbuf, dinv_t, x_t, sem):
    c = lax.axis_index("c")
    s = lax.axis_index("s")
    words = (NP * 128) // NS
    _fill_f32(zbuf, 2048, 0.0)
    tb = s * words

    def zloop(i, _):
        pltpu.sync_copy(zbuf, c_sh.at[pl.ds(tb + i * 2048, 2048)])
        return 0
    lax.fori_loop(0, words // 2048, zloop, 0)
    pltpu.sync_copy(dinv_hbm, dinv_t)
    pltpu.sync_copy(x_hbm, x_t)
    plsc.subcore_barrier()
    base = (c * NS + s) * EPW

    def chunk(j, _):
        b = base + j * _C_CH
        pltpu.sync_copy(src_hbm.at[pl.ds(b, _C_CH)], srcv)
        pltpu.sync_copy(dst_hbm.at[pl.ds(b, _C_CH)], dstv)

        def vec(i, _):
            sl = pl.ds(i * 16, 16)
            sv = srcv[sl]
            dv = dstv[sl]
            dsg = plsc.load_gather(dinv_t, [sv])
            ddg = plsc.load_gather(dinv_t, [dv])
            xg = plsc.load_gather(x_t, [sv])
            vv[sl] = dsg * ddg
            fv[sl] = dv * 128 + xg
            return 0
        lax.fori_loop(0, _C_CH // 16, vec, 0)
        pltpu.sync_copy(vv, c_sh.at[fv], add=True)
        return 0
    lax.fori_loop(0, EPW // _C_CH, chunk, 0)
    plsc.subcore_barrier()
    pltpu.sync_copy(c_sh.at[pl.ds(tb, words)], out_hbm.at[c, pl.ds(tb, words)])


@functools.lru_cache(maxsize=1)
def _cmat_call():
    return pl.kernel(
    _cmat_body,
    out_type=jax.ShapeDtypeStruct((NC, NP * 128), f32),
    mesh=plsc.VectorSubcoreMesh(core_axis_name="c", subcore_axis_name="s",
                                num_cores=NC, num_subcores=NS),
    compiler_params=pltpu.CompilerParams(needs_layout_passes=False),
    scratch_types=[
        pltpu.VMEM_SHARED((NP * 128,), f32),
        pltpu.VMEM((_C_CH,), i32),
        pltpu.VMEM((_C_CH,), i32),
        pltpu.VMEM((_C_CH,), f32),
        pltpu.VMEM((_C_CH,), i32),
        pltpu.VMEM((2048,), f32),
        pltpu.VMEM((NP,), f32),
        pltpu.VMEM((NP,), i32),
        pltpu.SemaphoreType.DMA,
    ],
)


_S_CH = 80


def _smat_body(src_hbm, dst_hbm, m_hbm, out_hbm,
               s_sh, srcv, dstv, rows, sem, srcv2, rows2, sem2):
    c = lax.axis_index("c")
    s = lax.axis_index("s")
    nrows = NP // NS

    def zbody(r, _):
        def inner(i, _):
            rows[r, pl.ds(i * 16, 16)] = jnp.zeros((16,), f32)
            return 0
        lax.fori_loop(0, 8, inner, 0)
        return 0
    lax.fori_loop(0, _S_CH, zbody, 0)

    def zcopy(i, _):
        pltpu.sync_copy(rows, s_sh.at[pl.ds(s * nrows + i * _S_CH, _S_CH)])
        return 0
    lax.fori_loop(0, nrows // _S_CH, zcopy, 0)
    rem = nrows % _S_CH
    if rem:
        pltpu.sync_copy(rows.at[pl.ds(0, rem)],
                        s_sh.at[pl.ds(s * nrows + nrows - rem, rem)])
    plsc.subcore_barrier()
    base = (c * NS + s) * EPW

    nch = EPW // _S_CH

    pltpu.sync_copy(src_hbm.at[pl.ds(base, _S_CH)], srcv)
    pltpu.async_copy(m_hbm.at[srcv], rows, sem)

    def chunk_pair(p, _):
        j0 = 2 * p
        pltpu.sync_copy(src_hbm.at[pl.ds(base + (j0 + 1) * _S_CH, _S_CH)],
                        srcv2)
        pltpu.async_copy(m_hbm.at[srcv2], rows2, sem2)
        pltpu.make_async_copy(m_hbm.at[srcv], rows, sem).wait()
        pltpu.sync_copy(dst_hbm.at[pl.ds(base + j0 * _S_CH, _S_CH)], dstv)
        pltpu.sync_copy(rows, s_sh.at[dstv], add=True)
        pltpu.sync_copy(src_hbm.at[pl.ds(base + (j0 + 2) * _S_CH, _S_CH)],
                        srcv)
        pltpu.async_copy(m_hbm.at[srcv], rows, sem)
        pltpu.make_async_copy(m_hbm.at[srcv2], rows2, sem2).wait()
        pltpu.sync_copy(dst_hbm.at[pl.ds(base + (j0 + 1) * _S_CH, _S_CH)],
                        dstv)
        pltpu.sync_copy(rows2, s_sh.at[dstv], add=True)
        return 0
    lax.fori_loop(0, nch // 2, chunk_pair, 0)
    pltpu.make_async_copy(m_hbm.at[srcv], rows, sem).wait()
    pltpu.sync_copy(dst_hbm.at[pl.ds(base + (nch - 1) * _S_CH, _S_CH)], dstv)
    pltpu.sync_copy(rows, s_sh.at[dstv], add=True)
    plsc.subcore_barrier()
    pltpu.sync_copy(s_sh.at[pl.ds(s * nrows, nrows)],
                    out_hbm.at[c, pl.ds(s * nrows, nrows)])


@functools.lru_cache(maxsize=1)
def _smat_call():
    return pl.kernel(
    _smat_body,
    out_type=jax.ShapeDtypeStruct((NC, NP, 128), f32),
    mesh=plsc.VectorSubcoreMesh(core_axis_name="c", subcore_axis_name="s",
                                num_cores=NC, num_subcores=NS),
    scratch_types=[
        pltpu.VMEM_SHARED((NP, 128), f32),
        pltpu.VMEM((_S_CH,), i32),
        pltpu.VMEM((_S_CH,), i32),
        pltpu.VMEM((_S_CH, 128), f32),
        pltpu.SemaphoreType.DMA,
        pltpu.VMEM((_S_CH,), i32),
        pltpu.VMEM((_S_CH, 128), f32),
        pltpu.SemaphoreType.DMA,
    ],
)


def _dinv_body(degp_ref, dinv_ref):
    d = degp_ref[0] + degp_ref[1] + 1.0
    dinv_ref[...] = lax.rsqrt(d)


def _tc_dinv(degp):
    return pl.pallas_call(
        _dinv_body,
        out_shape=jax.ShapeDtypeStruct((NP // 128, 128), f32),
    )(degp.reshape(NC, NP // 128, 128))


_BN = 1280
_NB = NP // _BN


def _dense_body(cp_ref, x_ref, dinv_ref, w1_ref, b1_ref, w2_ref, m_ref):
    cmat = cp_ref[0] + cp_ref[1]
    dinv = dinv_ref[...]
    oh = (x_ref[...] == lax.broadcasted_iota(i32, (1, 128), 1)).astype(f32)
    cmat = cmat + oh * (dinv * dinv)
    h1 = jnp.maximum(
        jnp.dot(cmat, w1_ref[...], preferred_element_type=f32) + b1_ref[...],
        0.0)
    m_ref[...] = jnp.dot(h1, w2_ref[...], preferred_element_type=f32) * dinv


def _tc_dense(cp, x_col, dinv_col, W1, b1, W2):
    return pl.pallas_call(
        _dense_body,
        grid=(_NB,),
        in_specs=[
            pl.BlockSpec((NC, _BN, 128), lambda i: (0, i, 0)),
            pl.BlockSpec((_BN, 1), lambda i: (i, 0)),
            pl.BlockSpec((_BN, 1), lambda i: (i, 0)),
            pl.BlockSpec((128, 128), lambda i: (0, 0)),
            pl.BlockSpec((1, 128), lambda i: (0, 0)),
            pl.BlockSpec((128, 128), lambda i: (0, 0)),
        ],
        out_specs=pl.BlockSpec((_BN, 128), lambda i: (i, 0)),
        out_shape=jax.ShapeDtypeStruct((NP, 128), f32),
    )(cp, x_col, dinv_col, W1, b1.reshape(1, H), W2)


def _h2_body(sp_ref, m_ref, dinv_ref, b2_ref, pw_ref, h2_ref, sc_ref):
    stot = sp_ref[0] + sp_ref[1] + m_ref[...]
    h2 = jnp.maximum(dinv_ref[...] * stot + b2_ref[...], 0.0)
    h2_ref[...] = h2
    pw = pw_ref[...]
    inv_norm = lax.rsqrt(jnp.sum(pw * pw))
    sc_ref[...] = jnp.dot(h2, pw, preferred_element_type=f32) * inv_norm


def _tc_h2(sp, m, dinv_col, b2, pool_w):
    return pl.pallas_call(
        _h2_body,
        grid=(_NB,),
        in_specs=[
            pl.BlockSpec((NC, _BN, 128), lambda i: (0, i, 0)),
            pl.BlockSpec((_BN, 128), lambda i: (i, 0)),
            pl.BlockSpec((_BN, 1), lambda i: (i, 0)),
            pl.BlockSpec((1, 128), lambda i: (0, 0)),
            pl.BlockSpec((128, 1), lambda i: (0, 0)),
        ],
        out_specs=[
            pl.BlockSpec((_BN, 128), lambda i: (i, 0)),
            pl.BlockSpec((_BN, 1), lambda i: (i, 0)),
        ],
        out_shape=[
            jax.ShapeDtypeStruct((NP, 128), f32),
            jax.ShapeDtypeStruct((NP, 1), f32),
        ],
    )(sp, m, dinv_col, b2.reshape(1, H), pool_w.reshape(H, 1))


_BJ = 512


def _pool_body(sc_col_ref, b_col_ref, sc_row_ref, b_row_ref, h2_ref, out_ref):
    pid = pl.program_id(0)
    s_col = sc_col_ref[...]
    b_col = b_col_ref[...]
    idx_col = pid * _BN + lax.broadcasted_iota(i32, (_BN, 1), 0)

    g_col = lax.broadcasted_iota(i32, (G, 1), 0)
    ind_full = (b_row_ref[...] == g_col).astype(f32)
    counts = jnp.sum(ind_full, axis=1, keepdims=True)
    kk = jnp.ceil(RATIO * counts)
    recip = 1.0 / jnp.maximum(kk, 1.0)

    def jchunk(j, acc):
        sl = pl.ds(j * _BJ, _BJ)
        s_row = sc_row_ref[:, sl]
        b_row = b_row_ref[:, sl]
        idx_row = j * _BJ + lax.broadcasted_iota(i32, (1, _BJ), 1)
        beat = (s_row > s_col) | ((s_row == s_col) & (idx_row < idx_col))
        cmp = ((b_row == b_col) & beat).astype(f32)
        return acc + jnp.sum(cmp, axis=1, keepdims=True)
    rank = lax.fori_loop(0, NP // _BJ, jchunk, jnp.zeros((_BN, 1), f32))

    ind_i = (b_col == lax.broadcasted_iota(i32, (1, G), 1)).astype(f32)
    k_node = jnp.dot(ind_i, kk, preferred_element_type=f32)
    r_node = jnp.dot(ind_i, recip, preferred_element_type=f32)
    keep = (rank < k_node).astype(f32)
    gate = jnp.tanh(s_col)
    p = h2_ref[...] * (gate * keep * r_node)

    b_row_blk = b_row_ref[:, pl.ds(pid * _BN, _BN)]
    ind_g = (g_col == b_row_blk).astype(f32)
    part = jnp.dot(ind_g, p, preferred_element_type=f32)

    @pl.when(pid == 0)
    def _():
        out_ref[...] = jnp.zeros((G, 128), f32)
    out_ref[...] += part


def _tc_pool(score, batch_col, h2):
    return pl.pallas_call(
        _pool_body,
        grid=(_NB,),
        in_specs=[
            pl.BlockSpec((_BN, 1), lambda i: (i, 0)),
            pl.BlockSpec((_BN, 1), lambda i: (i, 0)),
            pl.BlockSpec((1, NP), lambda i: (0, 0)),
            pl.BlockSpec((1, NP), lambda i: (0, 0)),
            pl.BlockSpec((_BN, 128), lambda i: (i, 0)),
        ],
        out_specs=pl.BlockSpec((G, 128), lambda i: (0, 0)),
        out_shape=jax.ShapeDtypeStruct((G, 128), f32),
    )(score, batch_col, score.reshape(1, NP), batch_col.reshape(1, NP), h2)


def kernel(x, edge_index, batch, W1, b1, W2, b2, pool_w):
    src = edge_index[0].astype(i32)
    dst = edge_index[1].astype(i32)
    pad = NP - N
    x_p = jnp.concatenate([x.astype(i32), jnp.zeros((pad,), i32)])
    batch_p = jnp.concatenate([batch.astype(i32), jnp.full((pad,), G, i32)])

    degp = _deg_call()(dst)
    dinv = _tc_dinv(degp).reshape(NP)
    cp = _cmat_call()(src, dst, x_p, dinv).reshape(NC, NP, 128)
    m = _tc_dense(cp, x_p.reshape(NP, 1), dinv.reshape(NP, 1), W1, b1, W2)
    sp = _smat_call()(src, dst, m)
    h2, score = _tc_h2(sp, m, dinv.reshape(NP, 1), b2, pool_w)
    emb = _tc_pool(score, batch_p.reshape(NP, 1), h2)
    return emb


# speedup vs baseline: 22.0186x; 1.4026x over previous
import functools
import jax
import jax.numpy as jnp
from jax import lax
from jax.experimental import pallas as pl
from jax.experimental.pallas import tpu as pltpu
from jax.experimental.pallas import tpu_sc as plsc

N = 10000
E = 320000
F_IN = 128
H = 128
G = 64
RATIO = 0.5

NP = 10240
NC = 2
NS = 16
EPS = E // NC
EPW = E // (NC * NS)

f32 = jnp.float32
i32 = jnp.int32


def _fill_f32(ref, n, val):
    def body(i, _):
        ref[pl.ds(i * 16, 16)] = jnp.full((16,), val, f32)
        return 0
    lax.fori_loop(0, n // 16, body, 0)


_DEG_CH = 2000


def _deg_body(dst_hbm, out_hbm, deg_sh, dstv, ones, zbuf, sem):
    c = lax.axis_index("c")
    s = lax.axis_index("s")
    rows = NP // NS
    _fill_f32(zbuf, rows, 0.0)
    pltpu.sync_copy(zbuf, deg_sh.at[pl.ds(s * rows, rows)])
    _fill_f32(ones, _DEG_CH, 1.0)
    plsc.subcore_barrier()
    base = (c * NS + s) * EPW

    def chunk(j, _):
        pltpu.sync_copy(dst_hbm.at[pl.ds(base + j * _DEG_CH, _DEG_CH)], dstv)
        pltpu.sync_copy(ones, deg_sh.at[dstv], add=True)
        return 0
    lax.fori_loop(0, EPW // _DEG_CH, chunk, 0)
    plsc.subcore_barrier()
    pltpu.sync_copy(deg_sh.at[pl.ds(s * rows, rows)],
                    out_hbm.at[c, pl.ds(s * rows, rows)])


@functools.lru_cache(maxsize=1)
def _deg_call():
    return pl.kernel(
    _deg_body,
    out_type=jax.ShapeDtypeStruct((NC, NP), f32),
    mesh=plsc.VectorSubcoreMesh(core_axis_name="c", subcore_axis_name="s",
                                num_cores=NC, num_subcores=NS),
    scratch_types=[
        pltpu.VMEM_SHARED((NP,), f32),
        pltpu.VMEM((_DEG_CH,), i32),
        pltpu.VMEM((_DEG_CH,), f32),
        pltpu.VMEM((NP // NS,), f32),
        pltpu.SemaphoreType.DMA,
    ],
)


_C_CH = 2000


def _cmat_body(src_hbm, dst_hbm, x_hbm, dinv_hbm, out_hbm,
               c_sh, srcv, dstv, vv, fv, zbuf, dinv_t, x_t, sem):
    c = lax.axis_index("c")
    s = lax.axis_index("s")
    words = (NP * 128) // NS
    _fill_f32(zbuf, 2048, 0.0)
    tb = s * words

    def zloop(i, _):
        pltpu.sync_copy(zbuf, c_sh.at[pl.ds(tb + i * 2048, 2048)])
        return 0
    lax.fori_loop(0, words // 2048, zloop, 0)
    pltpu.sync_copy(dinv_hbm, dinv_t)
    pltpu.sync_copy(x_hbm, x_t)
    plsc.subcore_barrier()
    base = (c * NS + s) * EPW

    def chunk(j, _):
        b = base + j * _C_CH
        pltpu.sync_copy(src_hbm.at[pl.ds(b, _C_CH)], srcv)
        pltpu.sync_copy(dst_hbm.at[pl.ds(b, _C_CH)], dstv)

        def vec(i, _):
            sl = pl.ds(i * 16, 16)
            sv = srcv[sl]
            dv = dstv[sl]
            dsg = plsc.load_gather(dinv_t, [sv])
            ddg = plsc.load_gather(dinv_t, [dv])
            xg = plsc.load_gather(x_t, [sv])
            vv[sl] = dsg * ddg
            fv[sl] = dv * 128 + xg
            return 0
        lax.fori_loop(0, _C_CH // 16, vec, 0)
        pltpu.sync_copy(vv, c_sh.at[fv], add=True)
        return 0
    lax.fori_loop(0, EPW // _C_CH, chunk, 0)
    plsc.subcore_barrier()
    pltpu.sync_copy(c_sh.at[pl.ds(tb, words)], out_hbm.at[c, pl.ds(tb, words)])


@functools.lru_cache(maxsize=1)
def _cmat_call():
    return pl.kernel(
    _cmat_body,
    out_type=jax.ShapeDtypeStruct((NC, NP * 128), f32),
    mesh=plsc.VectorSubcoreMesh(core_axis_name="c", subcore_axis_name="s",
                                num_cores=NC, num_subcores=NS),
    compiler_params=pltpu.CompilerParams(needs_layout_passes=False),
    scratch_types=[
        pltpu.VMEM_SHARED((NP * 128,), f32),
        pltpu.VMEM((_C_CH,), i32),
        pltpu.VMEM((_C_CH,), i32),
        pltpu.VMEM((_C_CH,), f32),
        pltpu.VMEM((_C_CH,), i32),
        pltpu.VMEM((2048,), f32),
        pltpu.VMEM((NP,), f32),
        pltpu.VMEM((NP,), i32),
        pltpu.SemaphoreType.DMA,
    ],
)


_S_CH = 80


def _smat_body(src_hbm, dst_hbm, m_hbm, out_hbm,
               s_sh, srcv, dstv, rows, sem, srcv2, rows2, sem2):
    c = lax.axis_index("c")
    s = lax.axis_index("s")
    nrows = NP // NS

    def zbody(r, _):
        def inner(i, _):
            rows[r, pl.ds(i * 16, 16)] = jnp.zeros((16,), f32)
            return 0
        lax.fori_loop(0, 8, inner, 0)
        return 0
    lax.fori_loop(0, _S_CH, zbody, 0)

    def zcopy(i, _):
        pltpu.sync_copy(rows, s_sh.at[pl.ds(s * nrows + i * _S_CH, _S_CH)])
        return 0
    lax.fori_loop(0, nrows // _S_CH, zcopy, 0)
    rem = nrows % _S_CH
    if rem:
        pltpu.sync_copy(rows.at[pl.ds(0, rem)],
                        s_sh.at[pl.ds(s * nrows + nrows - rem, rem)])
    plsc.subcore_barrier()
    base = (c * NS + s) * EPW

    nch = EPW // _S_CH

    pltpu.sync_copy(src_hbm.at[pl.ds(base, _S_CH)], srcv)
    pltpu.async_copy(m_hbm.at[srcv], rows, sem)

    def chunk_pair(p, _):
        j0 = 2 * p
        pltpu.sync_copy(src_hbm.at[pl.ds(base + (j0 + 1) * _S_CH, _S_CH)],
                        srcv2)
        pltpu.async_copy(m_hbm.at[srcv2], rows2, sem2)
        pltpu.make_async_copy(m_hbm.at[srcv], rows, sem).wait()
        pltpu.sync_copy(dst_hbm.at[pl.ds(base + j0 * _S_CH, _S_CH)], dstv)
        pltpu.sync_copy(rows, s_sh.at[dstv], add=True)
        pltpu.sync_copy(src_hbm.at[pl.ds(base + (j0 + 2) * _S_CH, _S_CH)],
                        srcv)
        pltpu.async_copy(m_hbm.at[srcv], rows, sem)
        pltpu.make_async_copy(m_hbm.at[srcv2], rows2, sem2).wait()
        pltpu.sync_copy(dst_hbm.at[pl.ds(base + (j0 + 1) * _S_CH, _S_CH)],
                        dstv)
        pltpu.sync_copy(rows2, s_sh.at[dstv], add=True)
        return 0
    lax.fori_loop(0, nch // 2, chunk_pair, 0)
    pltpu.make_async_copy(m_hbm.at[srcv], rows, sem).wait()
    pltpu.sync_copy(dst_hbm.at[pl.ds(base + (nch - 1) * _S_CH, _S_CH)], dstv)
    pltpu.sync_copy(rows, s_sh.at[dstv], add=True)
    plsc.subcore_barrier()
    pltpu.sync_copy(s_sh.at[pl.ds(s * nrows, nrows)],
                    out_hbm.at[c, pl.ds(s * nrows, nrows)])


@functools.lru_cache(maxsize=1)
def _smat_call():
    return pl.kernel(
    _smat_body,
    out_type=jax.ShapeDtypeStruct((NC, NP, 128), f32),
    mesh=plsc.VectorSubcoreMesh(core_axis_name="c", subcore_axis_name="s",
                                num_cores=NC, num_subcores=NS),
    scratch_types=[
        pltpu.VMEM_SHARED((NP, 128), f32),
        pltpu.VMEM((_S_CH,), i32),
        pltpu.VMEM((_S_CH,), i32),
        pltpu.VMEM((_S_CH, 128), f32),
        pltpu.SemaphoreType.DMA,
        pltpu.VMEM((_S_CH,), i32),
        pltpu.VMEM((_S_CH, 128), f32),
        pltpu.SemaphoreType.DMA,
    ],
)


def _dinv_body(degp_ref, dinv_ref):
    d = degp_ref[0] + degp_ref[1] + 1.0
    dinv_ref[...] = lax.rsqrt(d)


def _tc_dinv(degp):
    return pl.pallas_call(
        _dinv_body,
        out_shape=jax.ShapeDtypeStruct((NP // 128, 128), f32),
    )(degp.reshape(NC, NP // 128, 128))


_BN = 1280
_NB = NP // _BN


def _dense_body(cp_ref, x_ref, dinv_ref, w1_ref, b1_ref, w2_ref, m_ref):
    cmat = cp_ref[0] + cp_ref[1]
    dinv = dinv_ref[...]
    oh = (x_ref[...] == lax.broadcasted_iota(i32, (1, 128), 1)).astype(f32)
    cmat = cmat + oh * (dinv * dinv)
    h1 = jnp.maximum(
        jnp.dot(cmat, w1_ref[...], preferred_element_type=f32) + b1_ref[...],
        0.0)
    m_ref[...] = jnp.dot(h1, w2_ref[...], preferred_element_type=f32) * dinv


def _tc_dense(cp, x_col, dinv_col, W1, b1, W2):
    return pl.pallas_call(
        _dense_body,
        grid=(_NB,),
        in_specs=[
            pl.BlockSpec((NC, _BN, 128), lambda i: (0, i, 0)),
            pl.BlockSpec((_BN, 1), lambda i: (i, 0)),
            pl.BlockSpec((_BN, 1), lambda i: (i, 0)),
            pl.BlockSpec((128, 128), lambda i: (0, 0)),
            pl.BlockSpec((1, 128), lambda i: (0, 0)),
            pl.BlockSpec((128, 128), lambda i: (0, 0)),
        ],
        out_specs=pl.BlockSpec((_BN, 128), lambda i: (i, 0)),
        out_shape=jax.ShapeDtypeStruct((NP, 128), f32),
    )(cp, x_col, dinv_col, W1, b1.reshape(1, H), W2)


def _h2_body(sp_ref, m_ref, dinv_ref, b2_ref, pw_ref, h2_ref, sc_ref):
    stot = sp_ref[0] + sp_ref[1] + m_ref[...]
    h2 = jnp.maximum(dinv_ref[...] * stot + b2_ref[...], 0.0)
    h2_ref[...] = h2
    pw = pw_ref[...]
    inv_norm = lax.rsqrt(jnp.sum(pw * pw))
    sc_ref[...] = jnp.dot(h2, pw, preferred_element_type=f32) * inv_norm


def _tc_h2(sp, m, dinv_col, b2, pool_w):
    return pl.pallas_call(
        _h2_body,
        grid=(_NB,),
        in_specs=[
            pl.BlockSpec((NC, _BN, 128), lambda i: (0, i, 0)),
            pl.BlockSpec((_BN, 128), lambda i: (i, 0)),
            pl.BlockSpec((_BN, 1), lambda i: (i, 0)),
            pl.BlockSpec((1, 128), lambda i: (0, 0)),
            pl.BlockSpec((128, 1), lambda i: (0, 0)),
        ],
        out_specs=[
            pl.BlockSpec((_BN, 128), lambda i: (i, 0)),
            pl.BlockSpec((_BN, 1), lambda i: (i, 0)),
        ],
        out_shape=[
            jax.ShapeDtypeStruct((NP, 128), f32),
            jax.ShapeDtypeStruct((NP, 1), f32),
        ],
    )(sp, m, dinv_col, b2.reshape(1, H), pool_w.reshape(H, 1))


def _pool_body(sc_row_ref, b_row_ref, h2_ref, out_ref):
    u32 = jnp.uint32
    s_row = sc_row_ref[...]
    b_row = b_row_ref[...]

    g_col = lax.broadcasted_iota(i32, (G, 1), 0)
    ind = b_row == g_col
    counts = jnp.sum(ind.astype(i32), axis=1, keepdims=True)
    kk = (counts + 1) // 2
    recip = 1.0 / jnp.maximum(kk.astype(f32), 1.0)

    raw = lax.bitcast_convert_type(s_row, u32)
    hi = jnp.where(s_row < 0.0, ~raw, raw | u32(0x80000000))
    lo = ~lax.broadcasted_iota(u32, (1, NP), 1)

    def round_fn(keys, pre_mask_fn):
        def step(i, carry):
            pre, need = carry
            b = (31 - i).astype(u32)
            low_mask = (u32(2) << b) - u32(1)
            cand = pre_mask_fn() & (keys >= pre) & (keys <= (pre | low_mask))
            ones = cand & (((keys >> b) & u32(1)) == u32(1))
            cnt = jnp.sum(ones.astype(i32), axis=1, keepdims=True)
            take = cnt >= need
            need = jnp.where(take, need, need - cnt)
            pre = pre | jnp.where(take, u32(1) << b, u32(0))
            return pre, need
        return step

    pre_hi, need_a = lax.fori_loop(
        0, 32, round_fn(hi, lambda: b_row == g_col),
        (jnp.zeros((G, 1), u32), kk))
    pre_lo, _ = lax.fori_loop(
        0, 32, round_fn(lo, lambda: (b_row == g_col) & (hi == pre_hi)),
        (jnp.zeros((G, 1), u32), need_a))

    keep = ind & ((hi > pre_hi) | ((hi == pre_hi) & (lo >= pre_lo)))
    wmat = keep.astype(f32) * recip * jnp.tanh(s_row)
    out_ref[...] = jnp.dot(wmat, h2_ref[...], preferred_element_type=f32)


def _tc_pool(score, batch_col, h2):
    return pl.pallas_call(
        _pool_body,
        out_shape=jax.ShapeDtypeStruct((G, 128), f32),
    )(score.reshape(1, NP), batch_col.reshape(1, NP), h2)


def kernel(x, edge_index, batch, W1, b1, W2, b2, pool_w):
    src = edge_index[0].astype(i32)
    dst = edge_index[1].astype(i32)
    pad = NP - N
    x_p = jnp.concatenate([x.astype(i32), jnp.zeros((pad,), i32)])
    batch_p = jnp.concatenate([batch.astype(i32), jnp.full((pad,), G, i32)])

    degp = _deg_call()(dst)
    dinv = _tc_dinv(degp).reshape(NP)
    cp = _cmat_call()(src, dst, x_p, dinv).reshape(NC, NP, 128)
    m = _tc_dense(cp, x_p.reshape(NP, 1), dinv.reshape(NP, 1), W1, b1, W2)
    sp = _smat_call()(src, dst, m)
    h2, score = _tc_h2(sp, m, dinv.reshape(NP, 1), b2, pool_w)
    emb = _tc_pool(score, batch_p.reshape(NP, 1), h2)
    return emb


# speedup vs baseline: 22.4814x; 1.0210x over previous
import functools
import jax
import jax.numpy as jnp
from jax import lax
from jax.experimental import pallas as pl
from jax.experimental.pallas import tpu as pltpu
from jax.experimental.pallas import tpu_sc as plsc

N = 10000
E = 320000
F_IN = 128
H = 128
G = 64
RATIO = 0.5

NP = 10240
NC = 2
NS = 16
EPS = E // NC
EPW = E // (NC * NS)

f32 = jnp.float32
i32 = jnp.int32


def _fill_f32(ref, n, val):
    def body(i, _):
        ref[pl.ds(i * 16, 16)] = jnp.full((16,), val, f32)
        return 0
    lax.fori_loop(0, n // 16, body, 0)


_C_CH = 2000
_EPT = E // NS


def _front_body(ei_hbm, x_hbm, out_hbm, dinv_out,
                c_sh, deg_sh, dinv_sh,
                srcv, dstv, vv, fv, zbuf, dinv_t, x_t, sem):
    c = lax.axis_index("c")
    s = lax.axis_index("s")
    rows = NP // NS
    words = (NP * 128) // NS
    _fill_f32(zbuf, 2048, 0.0)
    tb = s * words

    def zloop(i, _):
        pltpu.sync_copy(zbuf, c_sh.at[pl.ds(tb + i * 2048, 2048)])
        return 0
    lax.fori_loop(0, words // 2048, zloop, 0)
    pltpu.sync_copy(zbuf.at[pl.ds(0, rows)], deg_sh.at[pl.ds(s * rows, rows)])
    _fill_f32(vv, _C_CH, 1.0)
    pltpu.sync_copy(x_hbm, x_t)
    plsc.subcore_barrier()

    dbase = s * _EPT

    def dchunk(j, _):
        pltpu.sync_copy(ei_hbm.at[pl.ds(E + dbase + j * _C_CH, _C_CH)], dstv)
        pltpu.sync_copy(vv, deg_sh.at[dstv], add=True)
        return 0
    lax.fori_loop(0, _EPT // _C_CH, dchunk, 0)
    plsc.subcore_barrier()

    pltpu.sync_copy(deg_sh.at[pl.ds(s * rows, rows)], vv.at[pl.ds(0, rows)])

    def newton(i, _):
        sl = pl.ds(i * 16, 16)
        xdeg = vv[sl] + 1.0
        seed = 0x5F3759DF - lax.shift_right_logical(
            plsc.bitcast(xdeg, i32), 1)
        y = plsc.bitcast(seed, f32)
        y = y * (1.5 - 0.5 * xdeg * y * y)
        y = y * (1.5 - 0.5 * xdeg * y * y)
        y = y * (1.5 - 0.5 * xdeg * y * y)
        vv[sl] = y
        return 0
    lax.fori_loop(0, rows // 16, newton, 0)
    pltpu.sync_copy(vv.at[pl.ds(0, rows)], dinv_sh.at[pl.ds(s * rows, rows)])
    plsc.subcore_barrier()
    pltpu.sync_copy(dinv_sh, dinv_t)

    @pl.when(jnp.logical_and(c == 0, s == 0))
    def _():
        pltpu.sync_copy(dinv_sh, dinv_out)

    base = (c * NS + s) * EPW

    def chunk(j, _):
        b = base + j * _C_CH
        pltpu.sync_copy(ei_hbm.at[pl.ds(b, _C_CH)], srcv)
        pltpu.sync_copy(ei_hbm.at[pl.ds(E + b, _C_CH)], dstv)

        def vec(i, _):
            sl = pl.ds(i * 16, 16)
            sv = srcv[sl]
            dv = dstv[sl]
            dsg = plsc.load_gather(dinv_t, [sv])
            ddg = plsc.load_gather(dinv_t, [dv])
            xg = plsc.load_gather(x_t, [sv])
            vv[sl] = dsg * ddg
            fv[sl] = dv * 128 + xg
            return 0
        lax.fori_loop(0, _C_CH // 16, vec, 0)
        pltpu.sync_copy(vv, c_sh.at[fv], add=True)
        return 0
    lax.fori_loop(0, EPW // _C_CH, chunk, 0)
    plsc.subcore_barrier()
    pltpu.sync_copy(c_sh.at[pl.ds(tb, words)], out_hbm.at[c, pl.ds(tb, words)])


@functools.lru_cache(maxsize=1)
def _front_call():
    return pl.kernel(
    _front_body,
    out_type=(jax.ShapeDtypeStruct((NC, NP * 128), f32),
              jax.ShapeDtypeStruct((NP,), f32)),
    mesh=plsc.VectorSubcoreMesh(core_axis_name="c", subcore_axis_name="s",
                                num_cores=NC, num_subcores=NS),
    compiler_params=pltpu.CompilerParams(needs_layout_passes=False),
    scratch_types=[
        pltpu.VMEM_SHARED((NP * 128,), f32),
        pltpu.VMEM_SHARED((NP,), f32),
        pltpu.VMEM_SHARED((NP,), f32),
        pltpu.VMEM((_C_CH,), i32),
        pltpu.VMEM((_C_CH,), i32),
        pltpu.VMEM((_C_CH,), f32),
        pltpu.VMEM((_C_CH,), i32),
        pltpu.VMEM((2048,), f32),
        pltpu.VMEM((NP,), f32),
        pltpu.VMEM((NP,), i32),
        pltpu.SemaphoreType.DMA,
    ],
)


_S_CH = 80


def _smat_body(ei_hbm, m_hbm, out_hbm,
               s_sh, srcv, dstv, rows, sem, srcv2, rows2, sem2):
    c = lax.axis_index("c")
    s = lax.axis_index("s")
    nrows = NP // NS

    def zbody(r, _):
        def inner(i, _):
            rows[r, pl.ds(i * 16, 16)] = jnp.zeros((16,), f32)
            return 0
        lax.fori_loop(0, 8, inner, 0)
        return 0
    lax.fori_loop(0, _S_CH, zbody, 0)

    def zcopy(i, _):
        pltpu.sync_copy(rows, s_sh.at[pl.ds(s * nrows + i * _S_CH, _S_CH)])
        return 0
    lax.fori_loop(0, nrows // _S_CH, zcopy, 0)
    rem = nrows % _S_CH
    if rem:
        pltpu.sync_copy(rows.at[pl.ds(0, rem)],
                        s_sh.at[pl.ds(s * nrows + nrows - rem, rem)])
    plsc.subcore_barrier()
    base = (c * NS + s) * EPW

    nch = EPW // _S_CH

    pltpu.sync_copy(ei_hbm.at[pl.ds(base, _S_CH)], srcv)
    pltpu.async_copy(m_hbm.at[srcv], rows, sem)

    def chunk_pair(p, _):
        j0 = 2 * p
        pltpu.sync_copy(ei_hbm.at[pl.ds(base + (j0 + 1) * _S_CH, _S_CH)],
                        srcv2)
        pltpu.async_copy(m_hbm.at[srcv2], rows2, sem2)
        pltpu.make_async_copy(m_hbm.at[srcv], rows, sem).wait()
        pltpu.sync_copy(ei_hbm.at[pl.ds(E + base + j0 * _S_CH, _S_CH)], dstv)
        pltpu.sync_copy(rows, s_sh.at[dstv], add=True)
        pltpu.sync_copy(ei_hbm.at[pl.ds(base + (j0 + 2) * _S_CH, _S_CH)],
                        srcv)
        pltpu.async_copy(m_hbm.at[srcv], rows, sem)
        pltpu.make_async_copy(m_hbm.at[srcv2], rows2, sem2).wait()
        pltpu.sync_copy(ei_hbm.at[pl.ds(E + base + (j0 + 1) * _S_CH, _S_CH)],
                        dstv)
        pltpu.sync_copy(rows2, s_sh.at[dstv], add=True)
        return 0
    lax.fori_loop(0, nch // 2, chunk_pair, 0)
    pltpu.make_async_copy(m_hbm.at[srcv], rows, sem).wait()
    pltpu.sync_copy(ei_hbm.at[pl.ds(E + base + (nch - 1) * _S_CH, _S_CH)], dstv)
    pltpu.sync_copy(rows, s_sh.at[dstv], add=True)
    plsc.subcore_barrier()
    pltpu.sync_copy(s_sh.at[pl.ds(s * nrows, nrows)],
                    out_hbm.at[c, pl.ds(s * nrows, nrows)])


@functools.lru_cache(maxsize=1)
def _smat_call():
    return pl.kernel(
    _smat_body,
    out_type=jax.ShapeDtypeStruct((NC, NP, 128), f32),
    mesh=plsc.VectorSubcoreMesh(core_axis_name="c", subcore_axis_name="s",
                                num_cores=NC, num_subcores=NS),
    scratch_types=[
        pltpu.VMEM_SHARED((NP, 128), f32),
        pltpu.VMEM((_S_CH,), i32),
        pltpu.VMEM((_S_CH,), i32),
        pltpu.VMEM((_S_CH, 128), f32),
        pltpu.SemaphoreType.DMA,
        pltpu.VMEM((_S_CH,), i32),
        pltpu.VMEM((_S_CH, 128), f32),
        pltpu.SemaphoreType.DMA,
    ],
)


_BN = 1280
_NB = NP // _BN


def _dense_body(cp_ref, x_ref, dinv_ref, w1_ref, b1_ref, w2_ref, m_ref):
    cmat = cp_ref[0] + cp_ref[1]
    dinv = dinv_ref[...]
    oh = (x_ref[...] == lax.broadcasted_iota(i32, (1, 128), 1)).astype(f32)
    cmat = cmat + oh * (dinv * dinv)
    h1 = jnp.maximum(
        jnp.dot(cmat, w1_ref[...], preferred_element_type=f32) + b1_ref[...],
        0.0)
    m_ref[...] = jnp.dot(h1, w2_ref[...], preferred_element_type=f32) * dinv


def _tc_dense(cp, x_col, dinv_col, W1, b1, W2):
    return pl.pallas_call(
        _dense_body,
        grid=(_NB,),
        in_specs=[
            pl.BlockSpec((NC, _BN, 128), lambda i: (0, i, 0)),
            pl.BlockSpec((_BN, 1), lambda i: (i, 0)),
            pl.BlockSpec((_BN, 1), lambda i: (i, 0)),
            pl.BlockSpec((128, 128), lambda i: (0, 0)),
            pl.BlockSpec((1, 128), lambda i: (0, 0)),
            pl.BlockSpec((128, 128), lambda i: (0, 0)),
        ],
        out_specs=pl.BlockSpec((_BN, 128), lambda i: (i, 0)),
        out_shape=jax.ShapeDtypeStruct((NP, 128), f32),
    )(cp, x_col, dinv_col, W1, b1.reshape(1, H), W2)


def _h2_body(sp_ref, m_ref, dinv_ref, b2_ref, pw_ref, h2_ref, sc_ref):
    stot = sp_ref[0] + sp_ref[1] + m_ref[...]
    h2 = jnp.maximum(dinv_ref[...] * stot + b2_ref[...], 0.0)
    h2_ref[...] = h2
    pw = pw_ref[...]
    inv_norm = lax.rsqrt(jnp.sum(pw * pw))
    sc_ref[...] = jnp.dot(h2, pw, preferred_element_type=f32) * inv_norm


def _tc_h2(sp, m, dinv_col, b2, pool_w):
    return pl.pallas_call(
        _h2_body,
        grid=(_NB,),
        in_specs=[
            pl.BlockSpec((NC, _BN, 128), lambda i: (0, i, 0)),
            pl.BlockSpec((_BN, 128), lambda i: (i, 0)),
            pl.BlockSpec((_BN, 1), lambda i: (i, 0)),
            pl.BlockSpec((1, 128), lambda i: (0, 0)),
            pl.BlockSpec((128, 1), lambda i: (0, 0)),
        ],
        out_specs=[
            pl.BlockSpec((_BN, 128), lambda i: (i, 0)),
            pl.BlockSpec((_BN, 1), lambda i: (i, 0)),
        ],
        out_shape=[
            jax.ShapeDtypeStruct((NP, 128), f32),
            jax.ShapeDtypeStruct((NP, 1), f32),
        ],
    )(sp, m, dinv_col, b2.reshape(1, H), pool_w.reshape(H, 1))


def _pool_body(sc_row_ref, b_row_ref, h2_ref, out_ref):
    u32 = jnp.uint32
    s_row = sc_row_ref[...]
    b_row = b_row_ref[...]

    g_col = lax.broadcasted_iota(i32, (G, 1), 0)
    ind = b_row == g_col
    counts = jnp.sum(ind.astype(i32), axis=1, keepdims=True)
    kk = (counts + 1) // 2
    recip = 1.0 / jnp.maximum(kk.astype(f32), 1.0)

    raw = lax.bitcast_convert_type(s_row, u32)
    hi = jnp.where(s_row < 0.0, ~raw, raw | u32(0x80000000))
    lo = ~lax.broadcasted_iota(u32, (1, NP), 1)

    def round_fn(keys, pre_mask_fn):
        def step(i, carry):
            pre, need = carry
            b = (31 - i).astype(u32)
            low_mask = (u32(2) << b) - u32(1)
            cand = pre_mask_fn() & (keys >= pre) & (keys <= (pre | low_mask))
            ones = cand & (((keys >> b) & u32(1)) == u32(1))
            cnt = jnp.sum(ones.astype(i32), axis=1, keepdims=True)
            take = cnt >= need
            need = jnp.where(take, need, need - cnt)
            pre = pre | jnp.where(take, u32(1) << b, u32(0))
            return pre, need
        return step

    pre_hi, need_a = lax.fori_loop(
        0, 32, round_fn(hi, lambda: b_row == g_col),
        (jnp.zeros((G, 1), u32), kk))
    pre_lo, _ = lax.fori_loop(
        0, 32, round_fn(lo, lambda: (b_row == g_col) & (hi == pre_hi)),
        (jnp.zeros((G, 1), u32), need_a))

    keep = ind & ((hi > pre_hi) | ((hi == pre_hi) & (lo >= pre_lo)))
    wmat = keep.astype(f32) * recip * jnp.tanh(s_row)
    out_ref[...] = jnp.dot(wmat, h2_ref[...], preferred_element_type=f32)


def _tc_pool(score, batch_col, h2):
    return pl.pallas_call(
        _pool_body,
        out_shape=jax.ShapeDtypeStruct((G, 128), f32),
    )(score.reshape(1, NP), batch_col.reshape(1, NP), h2)


def kernel(x, edge_index, batch, W1, b1, W2, b2, pool_w):
    ei_flat = edge_index.astype(i32).reshape(2 * E)
    pad = NP - N
    x_p = jnp.concatenate([x.astype(i32), jnp.zeros((pad,), i32)])
    batch_p = jnp.concatenate([batch.astype(i32), jnp.full((pad,), G, i32)])

    cpf, dinv = _front_call()(ei_flat, x_p)
    cp = cpf.reshape(NC, NP, 128)
    m = _tc_dense(cp, x_p.reshape(NP, 1), dinv.reshape(NP, 1), W1, b1, W2)
    sp = _smat_call()(ei_flat, m)
    h2, score = _tc_h2(sp, m, dinv.reshape(NP, 1), b2, pool_w)
    emb = _tc_pool(score, batch_p.reshape(NP, 1), h2)
    return emb


# speedup vs baseline: 23.1704x; 1.0307x over previous
import functools
import jax
import jax.numpy as jnp
from jax import lax
from jax.experimental import pallas as pl
from jax.experimental.pallas import tpu as pltpu
from jax.experimental.pallas import tpu_sc as plsc

N = 10000
E = 320000
F_IN = 128
H = 128
G = 64
RATIO = 0.5

NP = 10240
NC = 2
NS = 16
EPS = E // NC
EPW = E // (NC * NS)

f32 = jnp.float32
i32 = jnp.int32


def _fill_f32(ref, n, val):
    def body(i, _):
        ref[pl.ds(i * 16, 16)] = jnp.full((16,), val, f32)
        return 0
    lax.fori_loop(0, n // 16, body, 0)


_C_CH = 2000
_EPT = E // NS


def _front_body(ei_hbm, x_hbm, out0_hbm, out1_hbm, dinv_out,
                c_sh, deg_sh, dinv_sh,
                srcv, dstv, vv, fv, zbuf, dinv_t, x_t, sem):
    c = lax.axis_index("c")
    s = lax.axis_index("s")
    rows = NP // NS
    words = (NP * 128) // NS
    _fill_f32(zbuf, 2048, 0.0)
    tb = s * words

    def zloop(i, _):
        pltpu.sync_copy(zbuf, c_sh.at[pl.ds(tb + i * 2048, 2048)])
        return 0
    lax.fori_loop(0, words // 2048, zloop, 0)
    pltpu.sync_copy(zbuf.at[pl.ds(0, rows)], deg_sh.at[pl.ds(s * rows, rows)])
    _fill_f32(vv, _C_CH, 1.0)
    pltpu.sync_copy(x_hbm, x_t)
    plsc.subcore_barrier()

    dbase = s * _EPT

    def dchunk(j, _):
        pltpu.sync_copy(ei_hbm.at[pl.ds(E + dbase + j * _C_CH, _C_CH)], dstv)
        pltpu.sync_copy(vv, deg_sh.at[dstv], add=True)
        return 0
    lax.fori_loop(0, _EPT // _C_CH, dchunk, 0)
    plsc.subcore_barrier()

    pltpu.sync_copy(deg_sh.at[pl.ds(s * rows, rows)], vv.at[pl.ds(0, rows)])

    def newton(i, _):
        sl = pl.ds(i * 16, 16)
        xdeg = vv[sl] + 1.0
        seed = 0x5F3759DF - lax.shift_right_logical(
            plsc.bitcast(xdeg, i32), 1)
        y = plsc.bitcast(seed, f32)
        y = y * (1.5 - 0.5 * xdeg * y * y)
        y = y * (1.5 - 0.5 * xdeg * y * y)
        y = y * (1.5 - 0.5 * xdeg * y * y)
        vv[sl] = y
        return 0
    lax.fori_loop(0, rows // 16, newton, 0)
    pltpu.sync_copy(vv.at[pl.ds(0, rows)], dinv_sh.at[pl.ds(s * rows, rows)])
    plsc.subcore_barrier()
    pltpu.sync_copy(dinv_sh, dinv_t)

    @pl.when(jnp.logical_and(c == 0, s == 0))
    def _():
        pltpu.sync_copy(dinv_sh, dinv_out)

    base = (c * NS + s) * EPW

    def chunk(j, _):
        b = base + j * _C_CH
        pltpu.sync_copy(ei_hbm.at[pl.ds(b, _C_CH)], srcv)
        pltpu.sync_copy(ei_hbm.at[pl.ds(E + b, _C_CH)], dstv)

        def vec(i, _):
            sl = pl.ds(i * 16, 16)
            sv = srcv[sl]
            dv = dstv[sl]
            dsg = plsc.load_gather(dinv_t, [sv])
            ddg = plsc.load_gather(dinv_t, [dv])
            xg = plsc.load_gather(x_t, [sv])
            vv[sl] = dsg * ddg
            fv[sl] = dv * 128 + xg
            return 0
        lax.fori_loop(0, _C_CH // 16, vec, 0)
        pltpu.sync_copy(vv, c_sh.at[fv], add=True)
        return 0
    lax.fori_loop(0, EPW // _C_CH, chunk, 0)
    plsc.subcore_barrier()

    @pl.when(c == 0)
    def _():
        pltpu.sync_copy(c_sh.at[pl.ds(tb, words)], out0_hbm.at[pl.ds(tb, words)])

    @pl.when(c == 1)
    def _():
        pltpu.sync_copy(c_sh.at[pl.ds(tb, words)], out1_hbm.at[pl.ds(tb, words)])


@functools.lru_cache(maxsize=1)
def _front_call():
    return pl.kernel(
    _front_body,
    out_type=(jax.ShapeDtypeStruct((NP * 128,), f32),
              jax.ShapeDtypeStruct((NP * 128,), f32),
              jax.ShapeDtypeStruct((NP,), f32)),
    mesh=plsc.VectorSubcoreMesh(core_axis_name="c", subcore_axis_name="s",
                                num_cores=NC, num_subcores=NS),
    compiler_params=pltpu.CompilerParams(needs_layout_passes=False),
    scratch_types=[
        pltpu.VMEM_SHARED((NP * 128,), f32),
        pltpu.VMEM_SHARED((NP,), f32),
        pltpu.VMEM_SHARED((NP,), f32),
        pltpu.VMEM((_C_CH,), i32),
        pltpu.VMEM((_C_CH,), i32),
        pltpu.VMEM((_C_CH,), f32),
        pltpu.VMEM((_C_CH,), i32),
        pltpu.VMEM((2048,), f32),
        pltpu.VMEM((NP,), f32),
        pltpu.VMEM((NP,), i32),
        pltpu.SemaphoreType.DMA,
    ],
)


_S_CH = 80


def _smat_body(ei_hbm, m_hbm, out_hbm,
               s_sh, srcv, dstv, rows, sem, srcv2, rows2, sem2):
    c = lax.axis_index("c")
    s = lax.axis_index("s")
    nrows = NP // NS

    def zbody(r, _):
        def inner(i, _):
            rows[r, pl.ds(i * 16, 16)] = jnp.zeros((16,), f32)
            return 0
        lax.fori_loop(0, 8, inner, 0)
        return 0
    lax.fori_loop(0, _S_CH, zbody, 0)

    def zcopy(i, _):
        pltpu.sync_copy(rows, s_sh.at[pl.ds(s * nrows + i * _S_CH, _S_CH)])
        return 0
    lax.fori_loop(0, nrows // _S_CH, zcopy, 0)
    rem = nrows % _S_CH
    if rem:
        pltpu.sync_copy(rows.at[pl.ds(0, rem)],
                        s_sh.at[pl.ds(s * nrows + nrows - rem, rem)])
    plsc.subcore_barrier()
    base = (c * NS + s) * EPW

    nch = EPW // _S_CH

    pltpu.sync_copy(ei_hbm.at[pl.ds(base, _S_CH)], srcv)
    pltpu.async_copy(m_hbm.at[srcv], rows, sem)

    def chunk_pair(p, _):
        j0 = 2 * p
        pltpu.sync_copy(ei_hbm.at[pl.ds(base + (j0 + 1) * _S_CH, _S_CH)],
                        srcv2)
        pltpu.async_copy(m_hbm.at[srcv2], rows2, sem2)
        pltpu.make_async_copy(m_hbm.at[srcv], rows, sem).wait()
        pltpu.sync_copy(ei_hbm.at[pl.ds(E + base + j0 * _S_CH, _S_CH)], dstv)
        pltpu.sync_copy(rows, s_sh.at[dstv], add=True)
        pltpu.sync_copy(ei_hbm.at[pl.ds(base + (j0 + 2) * _S_CH, _S_CH)],
                        srcv)
        pltpu.async_copy(m_hbm.at[srcv], rows, sem)
        pltpu.make_async_copy(m_hbm.at[srcv2], rows2, sem2).wait()
        pltpu.sync_copy(ei_hbm.at[pl.ds(E + base + (j0 + 1) * _S_CH, _S_CH)],
                        dstv)
        pltpu.sync_copy(rows2, s_sh.at[dstv], add=True)
        return 0
    lax.fori_loop(0, nch // 2, chunk_pair, 0)
    pltpu.make_async_copy(m_hbm.at[srcv], rows, sem).wait()
    pltpu.sync_copy(ei_hbm.at[pl.ds(E + base + (nch - 1) * _S_CH, _S_CH)], dstv)
    pltpu.sync_copy(rows, s_sh.at[dstv], add=True)
    plsc.subcore_barrier()
    pltpu.sync_copy(s_sh.at[pl.ds(s * nrows, nrows)],
                    out_hbm.at[c, pl.ds(s * nrows, nrows)])


@functools.lru_cache(maxsize=1)
def _smat_call():
    return pl.kernel(
    _smat_body,
    out_type=jax.ShapeDtypeStruct((NC, NP, 128), f32),
    mesh=plsc.VectorSubcoreMesh(core_axis_name="c", subcore_axis_name="s",
                                num_cores=NC, num_subcores=NS),
    scratch_types=[
        pltpu.VMEM_SHARED((NP, 128), f32),
        pltpu.VMEM((_S_CH,), i32),
        pltpu.VMEM((_S_CH,), i32),
        pltpu.VMEM((_S_CH, 128), f32),
        pltpu.SemaphoreType.DMA,
        pltpu.VMEM((_S_CH,), i32),
        pltpu.VMEM((_S_CH, 128), f32),
        pltpu.SemaphoreType.DMA,
    ],
)


_BN = 1280
_NB = NP // _BN


def _dense_body(cp0_ref, cp1_ref, x_ref, dinv_ref, w1_ref, b1_ref, w2_ref,
                m_ref):
    cmat = cp0_ref[...] + cp1_ref[...]
    dinv = dinv_ref[...]
    oh = (x_ref[...] == lax.broadcasted_iota(i32, (1, 128), 1)).astype(f32)
    cmat = cmat + oh * (dinv * dinv)
    h1 = jnp.maximum(
        jnp.dot(cmat, w1_ref[...], preferred_element_type=f32) + b1_ref[...],
        0.0)
    m_ref[...] = jnp.dot(h1, w2_ref[...], preferred_element_type=f32) * dinv


def _tc_dense(cp0, cp1, x_col, dinv_col, W1, b1, W2):
    return pl.pallas_call(
        _dense_body,
        grid=(_NB,),
        in_specs=[
            pl.BlockSpec((_BN, 128), lambda i: (i, 0)),
            pl.BlockSpec((_BN, 128), lambda i: (i, 0)),
            pl.BlockSpec((_BN, 1), lambda i: (i, 0)),
            pl.BlockSpec((_BN, 1), lambda i: (i, 0)),
            pl.BlockSpec((128, 128), lambda i: (0, 0)),
            pl.BlockSpec((1, 128), lambda i: (0, 0)),
            pl.BlockSpec((128, 128), lambda i: (0, 0)),
        ],
        out_specs=pl.BlockSpec((_BN, 128), lambda i: (i, 0)),
        out_shape=jax.ShapeDtypeStruct((NP, 128), f32),
    )(cp0, cp1, x_col, dinv_col, W1, b1.reshape(1, H), W2)


def _h2_body(sp_ref, m_ref, dinv_ref, b2_ref, pw_ref, h2_ref, sc_ref):
    stot = sp_ref[0] + sp_ref[1] + m_ref[...]
    h2 = jnp.maximum(dinv_ref[...] * stot + b2_ref[...], 0.0)
    h2_ref[...] = h2
    pw = pw_ref[...]
    inv_norm = lax.rsqrt(jnp.sum(pw * pw))
    sc_ref[...] = jnp.dot(h2, pw, preferred_element_type=f32) * inv_norm


def _tc_h2(sp, m, dinv_col, b2, pool_w):
    return pl.pallas_call(
        _h2_body,
        grid=(_NB,),
        in_specs=[
            pl.BlockSpec((NC, _BN, 128), lambda i: (0, i, 0)),
            pl.BlockSpec((_BN, 128), lambda i: (i, 0)),
            pl.BlockSpec((_BN, 1), lambda i: (i, 0)),
            pl.BlockSpec((1, 128), lambda i: (0, 0)),
            pl.BlockSpec((128, 1), lambda i: (0, 0)),
        ],
        out_specs=[
            pl.BlockSpec((_BN, 128), lambda i: (i, 0)),
            pl.BlockSpec((_BN, 1), lambda i: (i, 0)),
        ],
        out_shape=[
            jax.ShapeDtypeStruct((NP, 128), f32),
            jax.ShapeDtypeStruct((NP, 1), f32),
        ],
    )(sp, m, dinv_col, b2.reshape(1, H), pool_w.reshape(H, 1))


def _pool_body(sc_row_ref, b_row_ref, h2_ref, out_ref):
    u32 = jnp.uint32
    s_row = sc_row_ref[...]
    b_row = b_row_ref[...]

    g_col = lax.broadcasted_iota(i32, (G, 1), 0)
    ind = b_row == g_col
    counts = jnp.sum(ind.astype(i32), axis=1, keepdims=True)
    kk = (counts + 1) // 2
    recip = 1.0 / jnp.maximum(kk.astype(f32), 1.0)

    raw = lax.bitcast_convert_type(s_row, u32)
    hi = jnp.where(s_row < 0.0, ~raw, raw | u32(0x80000000))
    lo = ~lax.broadcasted_iota(u32, (1, NP), 1)

    def round_fn(keys, pre_mask_fn):
        def step(i, carry):
            pre, need = carry
            b = (31 - i).astype(u32)
            low_mask = (u32(2) << b) - u32(1)
            cand = pre_mask_fn() & (keys >= pre) & (keys <= (pre | low_mask))
            ones = cand & (((keys >> b) & u32(1)) == u32(1))
            cnt = jnp.sum(ones.astype(i32), axis=1, keepdims=True)
            take = cnt >= need
            need = jnp.where(take, need, need - cnt)
            pre = pre | jnp.where(take, u32(1) << b, u32(0))
            return pre, need
        return step

    pre_hi, need_a = lax.fori_loop(
        0, 32, round_fn(hi, lambda: b_row == g_col),
        (jnp.zeros((G, 1), u32), kk))
    pre_lo, _ = lax.fori_loop(
        0, 32, round_fn(lo, lambda: (b_row == g_col) & (hi == pre_hi)),
        (jnp.zeros((G, 1), u32), need_a))

    keep = ind & ((hi > pre_hi) | ((hi == pre_hi) & (lo >= pre_lo)))
    wmat = keep.astype(f32) * recip * jnp.tanh(s_row)
    out_ref[...] = jnp.dot(wmat, h2_ref[...], preferred_element_type=f32)


def _tc_pool(score, batch_col, h2):
    return pl.pallas_call(
        _pool_body,
        out_shape=jax.ShapeDtypeStruct((G, 128), f32),
    )(score.reshape(1, NP), batch_col.reshape(1, NP), h2)


def kernel(x, edge_index, batch, W1, b1, W2, b2, pool_w):
    ei_flat = edge_index.astype(i32).reshape(2 * E)
    pad = NP - N
    x_p = jnp.concatenate([x.astype(i32), jnp.zeros((pad,), i32)])
    batch_p = jnp.concatenate([batch.astype(i32), jnp.full((pad,), G, i32)])

    cp0f, cp1f, dinv = _front_call()(ei_flat, x_p)
    m = _tc_dense(cp0f.reshape(NP, 128), cp1f.reshape(NP, 128),
                  x_p.reshape(NP, 1), dinv.reshape(NP, 1), W1, b1, W2)
    sp = _smat_call()(ei_flat, m)
    h2, score = _tc_h2(sp, m, dinv.reshape(NP, 1), b2, pool_w)
    emb = _tc_pool(score, batch_p.reshape(NP, 1), h2)
    return emb


# speedup vs baseline: 25.5837x; 1.1042x over previous
import functools
import jax
import jax.numpy as jnp
from jax import lax
from jax.experimental import pallas as pl
from jax.experimental.pallas import tpu as pltpu
from jax.experimental.pallas import tpu_sc as plsc

N = 10000
E = 320000
F_IN = 128
H = 128
G = 64
RATIO = 0.5

NP = 10240
NC = 2
NS = 16
EPS = E // NC
EPW = E // (NC * NS)

f32 = jnp.float32
i32 = jnp.int32


def _fill_f32(ref, n, val):
    def body(i, _):
        ref[pl.ds(i * 16, 16)] = jnp.full((16,), val, f32)
        return 0
    lax.fori_loop(0, n // 16, body, 0)


_C_CH = 2000
_EPT = E // NS


def _front_body(ei_hbm, x_hbm, out0_hbm, out1_hbm, dinv_out,
                c_sh, deg_sh, dinv_sh,
                srcv, dstv, vv, fv, zbuf, dinv_t, x_t, sem):
    c = lax.axis_index("c")
    s = lax.axis_index("s")
    rows = NP // NS
    words = (NP * 128) // NS
    _fill_f32(zbuf, 2048, 0.0)
    tb = s * words

    def zloop(i, _):
        pltpu.sync_copy(zbuf, c_sh.at[pl.ds(tb + i * 2048, 2048)])
        return 0
    lax.fori_loop(0, words // 2048, zloop, 0)
    pltpu.sync_copy(zbuf.at[pl.ds(0, rows)], deg_sh.at[pl.ds(s * rows, rows)])
    _fill_f32(vv, _C_CH, 1.0)
    pltpu.sync_copy(x_hbm, x_t)
    plsc.subcore_barrier()

    dbase = s * _EPT

    def dchunk(j, _):
        pltpu.sync_copy(ei_hbm.at[pl.ds(E + dbase + j * _C_CH, _C_CH)], dstv)
        pltpu.sync_copy(vv, deg_sh.at[dstv], add=True)
        return 0
    lax.fori_loop(0, _EPT // _C_CH, dchunk, 0)
    plsc.subcore_barrier()

    pltpu.sync_copy(deg_sh.at[pl.ds(s * rows, rows)], vv.at[pl.ds(0, rows)])

    def newton(i, _):
        sl = pl.ds(i * 16, 16)
        xdeg = vv[sl] + 1.0
        seed = 0x5F3759DF - lax.shift_right_logical(
            plsc.bitcast(xdeg, i32), 1)
        y = plsc.bitcast(seed, f32)
        y = y * (1.5 - 0.5 * xdeg * y * y)
        y = y * (1.5 - 0.5 * xdeg * y * y)
        y = y * (1.5 - 0.5 * xdeg * y * y)
        vv[sl] = y
        return 0
    lax.fori_loop(0, rows // 16, newton, 0)
    pltpu.sync_copy(vv.at[pl.ds(0, rows)], dinv_sh.at[pl.ds(s * rows, rows)])
    plsc.subcore_barrier()
    pltpu.sync_copy(dinv_sh, dinv_t)

    @pl.when(jnp.logical_and(c == 0, s == 0))
    def _():
        pltpu.sync_copy(dinv_sh, dinv_out)

    base = (c * NS + s) * EPW

    def chunk(j, _):
        b = base + j * _C_CH
        pltpu.sync_copy(ei_hbm.at[pl.ds(b, _C_CH)], srcv)
        pltpu.sync_copy(ei_hbm.at[pl.ds(E + b, _C_CH)], dstv)

        def vec(i, _):
            sl = pl.ds(i * 16, 16)
            sv = srcv[sl]
            dv = dstv[sl]
            dsg = plsc.load_gather(dinv_t, [sv])
            ddg = plsc.load_gather(dinv_t, [dv])
            xg = plsc.load_gather(x_t, [sv])
            vv[sl] = dsg * ddg
            fv[sl] = dv * 128 + xg
            return 0
        lax.fori_loop(0, _C_CH // 16, vec, 0)
        pltpu.sync_copy(vv, c_sh.at[fv], add=True)
        return 0
    lax.fori_loop(0, EPW // _C_CH, chunk, 0)
    plsc.subcore_barrier()

    @pl.when(c == 0)
    def _():
        pltpu.sync_copy(c_sh.at[pl.ds(tb, words)], out0_hbm.at[pl.ds(tb, words)])

    @pl.when(c == 1)
    def _():
        pltpu.sync_copy(c_sh.at[pl.ds(tb, words)], out1_hbm.at[pl.ds(tb, words)])


@functools.lru_cache(maxsize=1)
def _front_call():
    return pl.kernel(
    _front_body,
    out_type=(jax.ShapeDtypeStruct((NP * 128,), f32),
              jax.ShapeDtypeStruct((NP * 128,), f32),
              jax.ShapeDtypeStruct((NP,), f32)),
    mesh=plsc.VectorSubcoreMesh(core_axis_name="c", subcore_axis_name="s",
                                num_cores=NC, num_subcores=NS),
    compiler_params=pltpu.CompilerParams(needs_layout_passes=False),
    scratch_types=[
        pltpu.VMEM_SHARED((NP * 128,), f32),
        pltpu.VMEM_SHARED((NP,), f32),
        pltpu.VMEM_SHARED((NP,), f32),
        pltpu.VMEM((_C_CH,), i32),
        pltpu.VMEM((_C_CH,), i32),
        pltpu.VMEM((_C_CH,), f32),
        pltpu.VMEM((_C_CH,), i32),
        pltpu.VMEM((2048,), f32),
        pltpu.VMEM((NP,), f32),
        pltpu.VMEM((NP,), i32),
        pltpu.SemaphoreType.DMA,
    ],
)


_S_CH = 80


def _smat_body(ei_hbm, m_hbm, out_hbm, s_sh,
               srcv0, dstv0, rows0, gs0, ss0,
               srcv1, dstv1, rows1, gs1, ss1,
               srcv2, dstv2, rows2, gs2, ss2):
    c = lax.axis_index("c")
    s = lax.axis_index("s")
    nrows = NP // NS

    def zbody(r, _):
        def inner(i, _):
            rows0[r, pl.ds(i * 16, 16)] = jnp.zeros((16,), f32)
            return 0
        lax.fori_loop(0, 8, inner, 0)
        return 0
    lax.fori_loop(0, _S_CH, zbody, 0)

    def zcopy(i, _):
        pltpu.sync_copy(rows0, s_sh.at[pl.ds(s * nrows + i * _S_CH, _S_CH)])
        return 0
    lax.fori_loop(0, nrows // _S_CH, zcopy, 0)
    plsc.subcore_barrier()
    base = (c * NS + s) * EPW
    nch = EPW // _S_CH

    srcs = (srcv0, srcv1, srcv2)
    dsts = (dstv0, dstv1, dstv2)
    rws = (rows0, rows1, rows2)
    gss = (gs0, gs1, gs2)
    sss = (ss0, ss1, ss2)

    def issue(b, j):
        @pl.when(j >= 3)
        def _():
            pltpu.make_async_copy(rws[b], s_sh.at[dsts[b]], sss[b]).wait()
        pltpu.sync_copy(ei_hbm.at[pl.ds(base + j * _S_CH, _S_CH)], srcs[b])
        pltpu.async_copy(m_hbm.at[srcs[b]], rws[b], gss[b])

    def consume(b, j):
        pltpu.make_async_copy(m_hbm.at[srcs[b]], rws[b], gss[b]).wait()
        pltpu.sync_copy(ei_hbm.at[pl.ds(E + base + j * _S_CH, _S_CH)],
                        dsts[b])
        pltpu.async_copy(rws[b], s_sh.at[dsts[b]], sss[b], add=True)

    pltpu.sync_copy(ei_hbm.at[pl.ds(base, _S_CH)], srcs[0])
    pltpu.async_copy(m_hbm.at[srcs[0]], rws[0], gss[0])

    def trip(p, _):
        for b in range(3):
            j = 3 * p + b

            @pl.when(j + 1 <= nch - 1)
            def _(b=b, j=j):
                issue((b + 1) % 3, j + 1)

            @pl.when(j <= nch - 2)
            def _(b=b, j=j):
                consume(b, j)
        return 0
    lax.fori_loop(0, (nch + 2) // 3, trip, 0)
    consume((nch - 1) % 3, nch - 1)
    for b in range(3):
        pltpu.make_async_copy(rws[b], s_sh.at[dsts[b]], sss[b]).wait()
    plsc.subcore_barrier()
    pltpu.sync_copy(s_sh.at[pl.ds(s * nrows, nrows)],
                    out_hbm.at[c, pl.ds(s * nrows, nrows)])


@functools.lru_cache(maxsize=1)
def _smat_call():
    return pl.kernel(
    _smat_body,
    out_type=jax.ShapeDtypeStruct((NC, NP, 128), f32),
    mesh=plsc.VectorSubcoreMesh(core_axis_name="c", subcore_axis_name="s",
                                num_cores=NC, num_subcores=NS),
    scratch_types=[
        pltpu.VMEM_SHARED((NP, 128), f32),
        pltpu.VMEM((_S_CH,), i32),
        pltpu.VMEM((_S_CH,), i32),
        pltpu.VMEM((_S_CH, 128), f32),
        pltpu.SemaphoreType.DMA,
        pltpu.SemaphoreType.DMA,
        pltpu.VMEM((_S_CH,), i32),
        pltpu.VMEM((_S_CH,), i32),
        pltpu.VMEM((_S_CH, 128), f32),
        pltpu.SemaphoreType.DMA,
        pltpu.SemaphoreType.DMA,
        pltpu.VMEM((_S_CH,), i32),
        pltpu.VMEM((_S_CH,), i32),
        pltpu.VMEM((_S_CH, 128), f32),
        pltpu.SemaphoreType.DMA,
        pltpu.SemaphoreType.DMA,
    ],
)


_BN = 1280
_NB = NP // _BN


def _dense_body(cp0_ref, cp1_ref, x_ref, dinv_ref, w1_ref, b1_ref, w2_ref,
                m_ref):
    cmat = cp0_ref[...] + cp1_ref[...]
    dinv = dinv_ref[...]
    oh = (x_ref[...] == lax.broadcasted_iota(i32, (1, 128), 1)).astype(f32)
    cmat = cmat + oh * (dinv * dinv)
    h1 = jnp.maximum(
        jnp.dot(cmat, w1_ref[...], preferred_element_type=f32) + b1_ref[...],
        0.0)
    m_ref[...] = jnp.dot(h1, w2_ref[...], preferred_element_type=f32) * dinv


def _tc_dense(cp0, cp1, x_col, dinv_col, W1, b1, W2):
    return pl.pallas_call(
        _dense_body,
        grid=(_NB,),
        in_specs=[
            pl.BlockSpec((_BN, 128), lambda i: (i, 0)),
            pl.BlockSpec((_BN, 128), lambda i: (i, 0)),
            pl.BlockSpec((_BN, 1), lambda i: (i, 0)),
            pl.BlockSpec((_BN, 1), lambda i: (i, 0)),
            pl.BlockSpec((128, 128), lambda i: (0, 0)),
            pl.BlockSpec((1, 128), lambda i: (0, 0)),
            pl.BlockSpec((128, 128), lambda i: (0, 0)),
        ],
        out_specs=pl.BlockSpec((_BN, 128), lambda i: (i, 0)),
        out_shape=jax.ShapeDtypeStruct((NP, 128), f32),
    )(cp0, cp1, x_col, dinv_col, W1, b1.reshape(1, H), W2)


def _h2_body(sp_ref, m_ref, dinv_ref, b2_ref, pw_ref, h2_ref, sc_ref):
    stot = sp_ref[0] + sp_ref[1] + m_ref[...]
    h2 = jnp.maximum(dinv_ref[...] * stot + b2_ref[...], 0.0)
    h2_ref[...] = h2
    pw = pw_ref[...]
    inv_norm = lax.rsqrt(jnp.sum(pw * pw))
    sc_ref[...] = jnp.dot(h2, pw, preferred_element_type=f32) * inv_norm


def _tc_h2(sp, m, dinv_col, b2, pool_w):
    return pl.pallas_call(
        _h2_body,
        grid=(_NB,),
        in_specs=[
            pl.BlockSpec((NC, _BN, 128), lambda i: (0, i, 0)),
            pl.BlockSpec((_BN, 128), lambda i: (i, 0)),
            pl.BlockSpec((_BN, 1), lambda i: (i, 0)),
            pl.BlockSpec((1, 128), lambda i: (0, 0)),
            pl.BlockSpec((128, 1), lambda i: (0, 0)),
        ],
        out_specs=[
            pl.BlockSpec((_BN, 128), lambda i: (i, 0)),
            pl.BlockSpec((_BN, 1), lambda i: (i, 0)),
        ],
        out_shape=[
            jax.ShapeDtypeStruct((NP, 128), f32),
            jax.ShapeDtypeStruct((NP, 1), f32),
        ],
    )(sp, m, dinv_col, b2.reshape(1, H), pool_w.reshape(H, 1))


def _pool_body(sc_row_ref, b_row_ref, h2_ref, out_ref):
    u32 = jnp.uint32
    s_row = sc_row_ref[...]
    b_row = b_row_ref[...]

    g_col = lax.broadcasted_iota(i32, (G, 1), 0)
    ind = b_row == g_col
    counts = jnp.sum(ind.astype(i32), axis=1, keepdims=True)
    kk = (counts + 1) // 2
    recip = 1.0 / jnp.maximum(kk.astype(f32), 1.0)

    raw = lax.bitcast_convert_type(s_row, u32)
    hi = jnp.where(s_row < 0.0, ~raw, raw | u32(0x80000000))
    lo = ~lax.broadcasted_iota(u32, (1, NP), 1)

    def round_fn(keys, pre_mask_fn):
        def step(i, carry):
            pre, need = carry
            b = (31 - i).astype(u32)
            low_mask = (u32(2) << b) - u32(1)
            cand = pre_mask_fn() & (keys >= pre) & (keys <= (pre | low_mask))
            ones = cand & (((keys >> b) & u32(1)) == u32(1))
            cnt = jnp.sum(ones.astype(i32), axis=1, keepdims=True)
            take = cnt >= need
            need = jnp.where(take, need, need - cnt)
            pre = pre | jnp.where(take, u32(1) << b, u32(0))
            return pre, need
        return step

    pre_hi, need_a = lax.fori_loop(
        0, 32, round_fn(hi, lambda: b_row == g_col),
        (jnp.zeros((G, 1), u32), kk))
    pre_lo, _ = lax.fori_loop(
        0, 32, round_fn(lo, lambda: (b_row == g_col) & (hi == pre_hi)),
        (jnp.zeros((G, 1), u32), need_a))

    keep = ind & ((hi > pre_hi) | ((hi == pre_hi) & (lo >= pre_lo)))
    wmat = keep.astype(f32) * recip * jnp.tanh(s_row)
    out_ref[...] = jnp.dot(wmat, h2_ref[...], preferred_element_type=f32)


def _tc_pool(score, batch_col, h2):
    return pl.pallas_call(
        _pool_body,
        out_shape=jax.ShapeDtypeStruct((G, 128), f32),
    )(score.reshape(1, NP), batch_col.reshape(1, NP), h2)


def kernel(x, edge_index, batch, W1, b1, W2, b2, pool_w):
    ei_flat = edge_index.astype(i32).reshape(2 * E)
    pad = NP - N
    x_p = jnp.concatenate([x.astype(i32), jnp.zeros((pad,), i32)])
    batch_p = jnp.concatenate([batch.astype(i32), jnp.full((pad,), G, i32)])

    cp0f, cp1f, dinv = _front_call()(ei_flat, x_p)
    m = _tc_dense(cp0f.reshape(NP, 128), cp1f.reshape(NP, 128),
                  x_p.reshape(NP, 1), dinv.reshape(NP, 1), W1, b1, W2)
    sp = _smat_call()(ei_flat, m)
    h2, score = _tc_h2(sp, m, dinv.reshape(NP, 1), b2, pool_w)
    emb = _tc_pool(score, batch_p.reshape(NP, 1), h2)
    return emb


# speedup vs baseline: 27.9912x; 1.0941x over previous
import functools
import jax
import jax.numpy as jnp
from jax import lax
from jax.experimental import pallas as pl
from jax.experimental.pallas import tpu as pltpu
from jax.experimental.pallas import tpu_sc as plsc

N = 10000
E = 320000
F_IN = 128
H = 128
G = 64
RATIO = 0.5

NP = 10240
NC = 2
NS = 16
EPS = E // NC
EPW = E // (NC * NS)

f32 = jnp.float32
i32 = jnp.int32


def _fill_f32(ref, n, val):
    def body(i, _):
        ref[pl.ds(i * 16, 16)] = jnp.full((16,), val, f32)
        return 0
    lax.fori_loop(0, n // 16, body, 0)


_C_CH = 2000
_EPT = E // NS


def _front_body(ei_hbm, x_hbm, out0_hbm, out1_hbm, dinv_out,
                c_sh, deg_sh, dinv_sh,
                srcv, dstv, vv, fv, zbuf, dinv_t, x_t, sem):
    c = lax.axis_index("c")
    s = lax.axis_index("s")
    rows = NP // NS
    words = (NP * 128) // NS
    _fill_f32(zbuf, 2048, 0.0)
    tb = s * words

    def zloop(i, _):
        pltpu.sync_copy(zbuf, c_sh.at[pl.ds(tb + i * 2048, 2048)])
        return 0
    lax.fori_loop(0, words // 2048, zloop, 0)
    pltpu.sync_copy(zbuf.at[pl.ds(0, rows)], deg_sh.at[pl.ds(s * rows, rows)])
    _fill_f32(vv, _C_CH, 1.0)
    pltpu.sync_copy(x_hbm, x_t)
    plsc.subcore_barrier()

    dbase = s * _EPT

    def dchunk(j, _):
        pltpu.sync_copy(ei_hbm.at[pl.ds(E + dbase + j * _C_CH, _C_CH)], dstv)
        pltpu.sync_copy(vv, deg_sh.at[dstv], add=True)
        return 0
    lax.fori_loop(0, _EPT // _C_CH, dchunk, 0)
    plsc.subcore_barrier()

    pltpu.sync_copy(deg_sh.at[pl.ds(s * rows, rows)], vv.at[pl.ds(0, rows)])

    def newton(i, _):
        sl = pl.ds(i * 16, 16)
        xdeg = vv[sl] + 1.0
        seed = 0x5F3759DF - lax.shift_right_logical(
            plsc.bitcast(xdeg, i32), 1)
        y = plsc.bitcast(seed, f32)
        y = y * (1.5 - 0.5 * xdeg * y * y)
        y = y * (1.5 - 0.5 * xdeg * y * y)
        y = y * (1.5 - 0.5 * xdeg * y * y)
        vv[sl] = y
        return 0
    lax.fori_loop(0, rows // 16, newton, 0)
    pltpu.sync_copy(vv.at[pl.ds(0, rows)], dinv_sh.at[pl.ds(s * rows, rows)])
    plsc.subcore_barrier()
    pltpu.sync_copy(dinv_sh, dinv_t)

    @pl.when(jnp.logical_and(c == 0, s == 0))
    def _():
        pltpu.sync_copy(dinv_sh, dinv_out)

    base = (c * NS + s) * EPW

    def chunk(j, _):
        b = base + j * _C_CH
        pltpu.sync_copy(ei_hbm.at[pl.ds(b, _C_CH)], srcv)
        pltpu.sync_copy(ei_hbm.at[pl.ds(E + b, _C_CH)], dstv)

        def vec(i, _):
            sl = pl.ds(i * 16, 16)
            sv = srcv[sl]
            dv = dstv[sl]
            dsg = plsc.load_gather(dinv_t, [sv])
            ddg = plsc.load_gather(dinv_t, [dv])
            xg = plsc.load_gather(x_t, [sv])
            vv[sl] = dsg * ddg
            fv[sl] = dv * 128 + xg
            return 0
        lax.fori_loop(0, _C_CH // 16, vec, 0)
        pltpu.sync_copy(vv, c_sh.at[fv], add=True)
        return 0
    lax.fori_loop(0, EPW // _C_CH, chunk, 0)
    plsc.subcore_barrier()

    @pl.when(c == 0)
    def _():
        pltpu.sync_copy(c_sh.at[pl.ds(tb, words)], out0_hbm.at[pl.ds(tb, words)])

    @pl.when(c == 1)
    def _():
        pltpu.sync_copy(c_sh.at[pl.ds(tb, words)], out1_hbm.at[pl.ds(tb, words)])


@functools.lru_cache(maxsize=1)
def _front_call():
    return pl.kernel(
    _front_body,
    out_type=(jax.ShapeDtypeStruct((NP * 128,), f32),
              jax.ShapeDtypeStruct((NP * 128,), f32),
              jax.ShapeDtypeStruct((NP,), f32)),
    mesh=plsc.VectorSubcoreMesh(core_axis_name="c", subcore_axis_name="s",
                                num_cores=NC, num_subcores=NS),
    compiler_params=pltpu.CompilerParams(needs_layout_passes=False),
    scratch_types=[
        pltpu.VMEM_SHARED((NP * 128,), f32),
        pltpu.VMEM_SHARED((NP,), f32),
        pltpu.VMEM_SHARED((NP,), f32),
        pltpu.VMEM((_C_CH,), i32),
        pltpu.VMEM((_C_CH,), i32),
        pltpu.VMEM((_C_CH,), f32),
        pltpu.VMEM((_C_CH,), i32),
        pltpu.VMEM((2048,), f32),
        pltpu.VMEM((NP,), f32),
        pltpu.VMEM((NP,), i32),
        pltpu.SemaphoreType.DMA,
    ],
)


_S_CH = 80


def _smat_body(ei_hbm, m_hbm, out_hbm, s_sh,
               srcv0, dstv0, rows0, gs0, ss0,
               srcv1, dstv1, rows1, gs1, ss1,
               srcv2, dstv2, rows2, gs2, ss2):
    c = lax.axis_index("c")
    s = lax.axis_index("s")
    nrows = NP // NS

    def zbody(r, _):
        def inner(i, _):
            rows0[r, pl.ds(i * 16, 16)] = jnp.zeros((16,), f32)
            return 0
        lax.fori_loop(0, 8, inner, 0)
        return 0
    lax.fori_loop(0, _S_CH, zbody, 0)

    def zcopy(i, _):
        pltpu.sync_copy(rows0, s_sh.at[pl.ds(s * nrows + i * _S_CH, _S_CH)])
        return 0
    lax.fori_loop(0, nrows // _S_CH, zcopy, 0)
    plsc.subcore_barrier()
    base = (c * NS + s) * EPW
    nch = EPW // _S_CH

    srcs = (srcv0, srcv1, srcv2)
    dsts = (dstv0, dstv1, dstv2)
    rws = (rows0, rows1, rows2)
    gss = (gs0, gs1, gs2)
    sss = (ss0, ss1, ss2)

    def issue(b, j):
        @pl.when(j >= 3)
        def _():
            pltpu.make_async_copy(rws[b], s_sh.at[dsts[b]], sss[b]).wait()
        pltpu.sync_copy(ei_hbm.at[pl.ds(base + j * _S_CH, _S_CH)], srcs[b])
        pltpu.async_copy(m_hbm.at[srcs[b]], rws[b], gss[b])

    def consume(b, j):
        pltpu.make_async_copy(m_hbm.at[srcs[b]], rws[b], gss[b]).wait()
        pltpu.sync_copy(ei_hbm.at[pl.ds(E + base + j * _S_CH, _S_CH)],
                        dsts[b])
        pltpu.async_copy(rws[b], s_sh.at[dsts[b]], sss[b], add=True)

    pltpu.sync_copy(ei_hbm.at[pl.ds(base, _S_CH)], srcs[0])
    pltpu.async_copy(m_hbm.at[srcs[0]], rws[0], gss[0])

    def trip(p, _):
        for b in range(3):
            j = 3 * p + b

            @pl.when(j + 1 <= nch - 1)
            def _(b=b, j=j):
                issue((b + 1) % 3, j + 1)

            @pl.when(j <= nch - 2)
            def _(b=b, j=j):
                consume(b, j)
        return 0
    lax.fori_loop(0, (nch + 2) // 3, trip, 0)
    consume((nch - 1) % 3, nch - 1)
    for b in range(3):
        pltpu.make_async_copy(rws[b], s_sh.at[dsts[b]], sss[b]).wait()
    plsc.subcore_barrier()
    pltpu.sync_copy(s_sh.at[pl.ds(s * nrows, nrows)],
                    out_hbm.at[c, pl.ds(s * nrows, nrows)])


@functools.lru_cache(maxsize=1)
def _smat_call():
    return pl.kernel(
    _smat_body,
    out_type=jax.ShapeDtypeStruct((NC, NP, 128), f32),
    mesh=plsc.VectorSubcoreMesh(core_axis_name="c", subcore_axis_name="s",
                                num_cores=NC, num_subcores=NS),
    scratch_types=[
        pltpu.VMEM_SHARED((NP, 128), f32),
        pltpu.VMEM((_S_CH,), i32),
        pltpu.VMEM((_S_CH,), i32),
        pltpu.VMEM((_S_CH, 128), f32),
        pltpu.SemaphoreType.DMA,
        pltpu.SemaphoreType.DMA,
        pltpu.VMEM((_S_CH,), i32),
        pltpu.VMEM((_S_CH,), i32),
        pltpu.VMEM((_S_CH, 128), f32),
        pltpu.SemaphoreType.DMA,
        pltpu.SemaphoreType.DMA,
        pltpu.VMEM((_S_CH,), i32),
        pltpu.VMEM((_S_CH,), i32),
        pltpu.VMEM((_S_CH, 128), f32),
        pltpu.SemaphoreType.DMA,
        pltpu.SemaphoreType.DMA,
    ],
)


_BN = 1280
_NB = NP // _BN


def _dense_body(cp0_ref, cp1_ref, x_ref, dinv_ref, w1_ref, b1_ref, w2_ref,
                m_ref):
    cmat = cp0_ref[...] + cp1_ref[...]
    dinv = dinv_ref[...]
    oh = (x_ref[...] == lax.broadcasted_iota(i32, (1, 128), 1)).astype(f32)
    cmat = cmat + oh * (dinv * dinv)
    h1 = jnp.maximum(
        jnp.dot(cmat, w1_ref[...], preferred_element_type=f32) + b1_ref[...],
        0.0)
    m_ref[...] = jnp.dot(h1, w2_ref[...], preferred_element_type=f32) * dinv


def _tc_dense(cp0, cp1, x_col, dinv_col, W1, b1, W2):
    return pl.pallas_call(
        _dense_body,
        grid=(_NB,),
        in_specs=[
            pl.BlockSpec((_BN, 128), lambda i: (i, 0)),
            pl.BlockSpec((_BN, 128), lambda i: (i, 0)),
            pl.BlockSpec((_BN, 1), lambda i: (i, 0)),
            pl.BlockSpec((_BN, 1), lambda i: (i, 0)),
            pl.BlockSpec((128, 128), lambda i: (0, 0)),
            pl.BlockSpec((1, 128), lambda i: (0, 0)),
            pl.BlockSpec((128, 128), lambda i: (0, 0)),
        ],
        out_specs=pl.BlockSpec((_BN, 128), lambda i: (i, 0)),
        out_shape=jax.ShapeDtypeStruct((NP, 128), f32),
    )(cp0, cp1, x_col, dinv_col, W1, b1.reshape(1, H), W2)


def _h2_body(sp_ref, m_ref, dinv_ref, b2_ref, pw_ref, h2_ref, sc_ref):
    stot = sp_ref[0] + sp_ref[1] + m_ref[...]
    h2 = jnp.maximum(dinv_ref[...] * stot + b2_ref[...], 0.0)
    h2_ref[...] = h2
    pw = pw_ref[...]
    inv_norm = lax.rsqrt(jnp.sum(pw * pw))
    sc_ref[...] = jnp.dot(h2, pw, preferred_element_type=f32) * inv_norm


def _tc_h2(sp, m, dinv_col, b2, pool_w):
    return pl.pallas_call(
        _h2_body,
        grid=(_NB,),
        in_specs=[
            pl.BlockSpec((NC, _BN, 128), lambda i: (0, i, 0)),
            pl.BlockSpec((_BN, 128), lambda i: (i, 0)),
            pl.BlockSpec((_BN, 1), lambda i: (i, 0)),
            pl.BlockSpec((1, 128), lambda i: (0, 0)),
            pl.BlockSpec((128, 1), lambda i: (0, 0)),
        ],
        out_specs=[
            pl.BlockSpec((_BN, 128), lambda i: (i, 0)),
            pl.BlockSpec((_BN, 1), lambda i: (i, 0)),
        ],
        out_shape=[
            jax.ShapeDtypeStruct((NP, 128), f32),
            jax.ShapeDtypeStruct((NP, 1), f32),
        ],
    )(sp, m, dinv_col, b2.reshape(1, H), pool_w.reshape(H, 1))


def _pool_body(sc_row_ref, b_row_ref, h2_ref, out_ref):
    u32 = jnp.uint32
    s_row = sc_row_ref[...]
    b_row = b_row_ref[...]

    g_col = lax.broadcasted_iota(i32, (G, 1), 0)
    ind = b_row == g_col
    counts = jnp.sum(ind.astype(i32), axis=1, keepdims=True)
    kk = (counts + 1) // 2
    recip = 1.0 / jnp.maximum(kk.astype(f32), 1.0)

    raw = lax.bitcast_convert_type(s_row, u32)
    score_u = jnp.where(s_row < 0.0, ~raw, raw | u32(0x80000000))
    bk = (u32(G) - b_row.astype(u32)) & u32(0x7F)
    hi = (bk << 25) | (score_u >> 7)
    lo = ((score_u & u32(0x7F)) << 25) | \
        ((u32(16383) - lax.broadcasted_iota(u32, (1, NP), 1)) << 11)

    tri = (lax.broadcasted_iota(i32, (G, G), 1) <
           lax.broadcasted_iota(i32, (G, G), 0)).astype(f32)
    start = jnp.dot(tri, counts.astype(f32), preferred_element_type=f32)
    need0 = start.astype(i32) + kk

    def round_fn(keys, pre_mask_fn=None):
        def step(i, carry):
            pre, need = carry
            b = (31 - i).astype(u32)
            low_mask = (u32(2) << b) - u32(1)
            cand = (keys >= pre) & (keys <= (pre | low_mask))
            if pre_mask_fn is not None:
                cand = cand & pre_mask_fn()
            ones = cand & (((keys >> b) & u32(1)) == u32(1))
            cnt = jnp.sum(ones.astype(i32), axis=1, keepdims=True)
            take = cnt >= need
            need = jnp.where(take, need, need - cnt)
            pre = pre | jnp.where(take, u32(1) << b, u32(0))
            return pre, need
        return step

    pre_hi, need_a = lax.fori_loop(
        0, 32, round_fn(hi),
        (jnp.zeros((G, 1), u32), need0))
    pre_lo, _ = lax.fori_loop(
        0, 21, round_fn(lo, lambda: hi == pre_hi),
        (jnp.zeros((G, 1), u32), need_a))

    keep = ind & ((hi > pre_hi) | ((hi == pre_hi) & (lo >= pre_lo)))
    wmat = keep.astype(f32) * recip * jnp.tanh(s_row)
    out_ref[...] = jnp.dot(wmat, h2_ref[...], preferred_element_type=f32)


def _tc_pool(score, batch_col, h2):
    return pl.pallas_call(
        _pool_body,
        out_shape=jax.ShapeDtypeStruct((G, 128), f32),
    )(score.reshape(1, NP), batch_col.reshape(1, NP), h2)


def kernel(x, edge_index, batch, W1, b1, W2, b2, pool_w):
    ei_flat = edge_index.astype(i32).reshape(2 * E)
    pad = NP - N
    x_p = jnp.concatenate([x.astype(i32), jnp.zeros((pad,), i32)])
    batch_p = jnp.concatenate([batch.astype(i32), jnp.full((pad,), G, i32)])

    cp0f, cp1f, dinv = _front_call()(ei_flat, x_p)
    m = _tc_dense(cp0f.reshape(NP, 128), cp1f.reshape(NP, 128),
                  x_p.reshape(NP, 1), dinv.reshape(NP, 1), W1, b1, W2)
    sp = _smat_call()(ei_flat, m)
    h2, score = _tc_h2(sp, m, dinv.reshape(NP, 1), b2, pool_w)
    emb = _tc_pool(score, batch_p.reshape(NP, 1), h2)
    return emb
